# baseline (speedup 1.0000x reference)
import functools

import jax
import jax.numpy as jnp
from jax import lax
from jax.experimental import pallas as pl
from jax.experimental.pallas import tpu as pltpu
from jax.experimental.pallas import tpu_sc as plsc

_NC = 2
_NS = 16
_CHUNKS = 4


def _proj_kernel(emb_ref, wt_ref, b_ref, p_ref):
    p_ref[...] = (
        jnp.dot(emb_ref[...], wt_ref[...], preferred_element_type=jnp.float32)
        + b_ref[...]
    )


def _compute_table(emb, wt, b2d):
    v = emb.shape[0]
    n = wt.shape[1]
    return pl.pallas_call(
        _proj_kernel,
        out_shape=jax.ShapeDtypeStruct((v, n), jnp.float32),
    )(emb, wt, b2d)


def _sc_gather_chunk(p, idx_flat, nb, spad):
    d = p.shape[2]
    sub = p.shape[1]
    nw = _NC * _NS
    bpw = nb // nw
    mesh = plsc.VectorSubcoreMesh(core_axis_name="c", subcore_axis_name="s")

    @functools.partial(
        pl.kernel,
        mesh=mesh,
        out_type=jax.ShapeDtypeStruct((nb, spad, sub, d), jnp.float32),
        scratch_types=[
            pltpu.VMEM((bpw * spad,), jnp.int32),
            pltpu.VMEM((2, spad, sub, d), jnp.float32),
            pltpu.SemaphoreType.DMA,
            pltpu.SemaphoreType.DMA,
            pltpu.SemaphoreType.DMA,
        ],
    )
    def k(p_hbm, idx_hbm, out_hbm, idx_v, rows_v, gsem, wsem0, wsem1):
        wid = lax.axis_index("s") * _NC + lax.axis_index("c")
        base = wid * bpw
        pltpu.sync_copy(idx_hbm.at[pl.ds(base * spad, bpw * spad)], idx_v)
        wsems = (wsem0, wsem1)

        def body(j2, carry):
            for t in (0, 1):
                j = 2 * j2 + t

                @pl.when(j2 >= 1)
                def _(t=t, j=j):
                    pltpu.make_async_copy(
                        rows_v.at[t], out_hbm.at[base + j - 2], wsems[t]
                    ).wait()

                pltpu.async_copy(
                    p_hbm.at[idx_v.at[pl.ds(j * spad, spad)]],
                    rows_v.at[t],
                    gsem,
                ).wait()
                pltpu.async_copy(
                    rows_v.at[t], out_hbm.at[base + j], wsems[t]
                )
            return carry

        lax.fori_loop(0, bpw // 2, body, 0)
        for t in (0, 1):
            pltpu.make_async_copy(
                rows_v.at[t], out_hbm.at[base + bpw - 2 + t], wsems[t]
            ).wait()

    return k(p, idx_flat)


def kernel(input_seq, emb_table, W, b):
    batch, seq = input_seq.shape
    vocab, dim = emb_table.shape
    dpad = 1024
    spad = 56
    idx_flat = jnp.pad(
        input_seq.astype(jnp.int32), ((0, 0), (0, spad - seq))
    ).reshape(-1)
    wtp = jnp.pad(W.T, ((0, 0), (0, dpad - vocab)))
    b2 = jnp.pad(b, (0, dpad - vocab)).reshape(1, dpad)
    p = _compute_table(emb_table, wtp, b2).reshape(vocab, 8, dpad // 8)
    nb = batch // _CHUNKS
    parts = []
    for c in range(_CHUNKS):
        pad_chunk = _sc_gather_chunk(
            p, lax.dynamic_slice_in_dim(idx_flat, c * nb * spad, nb * spad),
            nb, spad,
        )
        parts.append(
            pad_chunk.reshape(nb, spad, dpad)[:, :seq, :vocab]
        )
    return jnp.concatenate(parts, axis=0)


# speedup vs baseline: 2.6236x; 2.6236x over previous
import jax
import jax.numpy as jnp
from jax.experimental import pallas as pl


def _tc_kernel(idx_ref, emb_ref, wt_ref, b_ref, out_ref):
    rows = idx_ref.shape[0]
    vocab = emb_ref.shape[0]
    iota = jax.lax.broadcasted_iota(jnp.int32, (rows, vocab), 1)
    oh = (idx_ref[...] == iota).astype(jnp.float32)
    e = jnp.dot(oh, emb_ref[...], preferred_element_type=jnp.float32)
    y = jnp.dot(e, wt_ref[...], preferred_element_type=jnp.float32) + b_ref[...]
    bb = out_ref.shape[0]
    out_ref[...] = y.reshape(bb, out_ref.shape[1], out_ref.shape[2])


def kernel(input_seq, emb_table, W, b):
    batch, seq = input_seq.shape
    vocab, dim = emb_table.shape
    idx2 = input_seq.reshape(batch * seq, 1).astype(jnp.int32)
    wt = W.T
    b2 = b.reshape(1, vocab)
    bb = 8
    grid = (batch // bb,)
    return pl.pallas_call(
        _tc_kernel,
        grid=grid,
        in_specs=[
            pl.BlockSpec((bb * seq, 1), lambda i: (i, 0)),
            pl.BlockSpec((vocab, dim), lambda i: (0, 0)),
            pl.BlockSpec((dim, vocab), lambda i: (0, 0)),
            pl.BlockSpec((1, vocab), lambda i: (0, 0)),
        ],
        out_specs=pl.BlockSpec((bb, seq, vocab), lambda i: (i, 0, 0)),
        out_shape=jax.ShapeDtypeStruct((batch, seq, vocab), jnp.float32),
    )(idx2, emb_table, wt, b2)
